# Initial kernel scaffold; baseline (speedup 1.0000x reference)
import jax
import jax.numpy as jnp
from jax.experimental import pallas as pl


def kernel(X, eps1, W1a, b1a, g1a, be1a, W1b, b1b, g1b, be1b, eps2, W2a, b2a, g2a, be2a, W2b, b2b, g2b, be2b):
    raise NotImplementedError("write your pallas kernel here")


# speedup vs baseline: 1.6547x; 1.6547x over previous
import jax
import jax.numpy as jnp
from jax import lax
from jax.experimental import pallas as pl
from jax.experimental.pallas import tpu as pltpu

_B, _T, _N, _H = 8, 512, 200, 128

_DN_TT = (((0,), (0,)), ((), ()))
_DN_NT = (((1,), (1,)), ((), ()))
_DN_NN = (((1,), (0,)), ((), ()))


def _mm(a, b, dn):
    return lax.dot_general(a, b, dn, preferred_element_type=jnp.float32)


def _mm_hi(a, b, dn):
    return lax.dot_general(a, b, dn, precision=lax.Precision.HIGHEST,
                           preferred_element_type=jnp.float32)


def _body(X_ref, eps1_ref, W1a_ref, b1a_ref, g1a_ref, be1a_ref,
          W1b_ref, b1b_ref, g1b_ref, be1b_ref,
          eps2_ref, W2a_ref, b2a_ref, g2a_ref, be2a_ref,
          W2b_ref, b2b_ref, g2b_ref, be2b_ref, out_ref):
    eps1 = eps1_ref[0, 0]
    eps2 = eps2_ref[0, 0]

    vs, masks = [], []
    ii = lax.broadcasted_iota(jnp.int32, (_N, _N), 0)
    jj = lax.broadcasted_iota(jnp.int32, (_N, _N), 1)
    eye = (ii == jj).astype(jnp.float32)
    for b in range(_B):
        x = X_ref[b]
        xm = x - jnp.mean(x, axis=0, keepdims=True)
        c = _mm(xm, xm, _DN_TT) / (_T - 1)
        d = jnp.sum(c * eye, axis=0, keepdims=True)
        std = jnp.sqrt(d)
        denom = _mm_hi(std, std, _DN_TT)
        c = c / denom
        c = jnp.clip(c, -1.0, 1.0)
        c = jnp.where(jnp.isnan(c), 0.0, c)
        vs.append(jnp.abs(c))
        masks.append((c != 0.0).astype(jnp.float32))

    def gin(feats, eps, Wa, ba, ga, bea, Wb, bb, gb, beb):
        h1 = []
        for b in range(_B):
            agg = _mm(masks[b], feats[b], _DN_NN) + eps * feats[b]
            h1.append(_mm(agg, Wa, _DN_NT) + ba)
        inv_rows = 1.0 / (_B * _N)
        m = sum(jnp.sum(h, axis=0, keepdims=True) for h in h1) * inv_rows
        var = sum(jnp.sum((h - m) * (h - m), axis=0, keepdims=True)
                  for h in h1) * inv_rows
        scale = ga * lax.rsqrt(var + 1e-5)
        h1 = [jnp.maximum((h - m) * scale + bea, 0.0) for h in h1]
        h2 = [_mm(h, Wb, _DN_NT) + bb for h in h1]
        m2 = sum(jnp.sum(h, axis=0, keepdims=True) for h in h2) * inv_rows
        var2 = sum(jnp.sum((h - m2) * (h - m2), axis=0, keepdims=True)
                   for h in h2) * inv_rows
        scale2 = gb * lax.rsqrt(var2 + 1e-5)
        return [jnp.maximum((h - m2) * scale2 + beb, 0.0) for h in h2]

    x1 = gin(vs, eps1, W1a_ref[...], b1a_ref[...], g1a_ref[...], be1a_ref[...],
             W1b_ref[...], b1b_ref[...], g1b_ref[...], be1b_ref[...])
    x2 = gin(x1, eps2, W2a_ref[...], b2a_ref[...], g2a_ref[...], be2a_ref[...],
             W2b_ref[...], b2b_ref[...], g2b_ref[...], be2b_ref[...])
    for b in range(_B):
        out_ref[b, :, :] = x2[b]


def kernel(X, eps1, W1a, b1a, g1a, be1a, W1b, b1b, g1b, be1b,
           eps2, W2a, b2a, g2a, be2a, W2b, b2b, g2b, be2b):
    r = lambda v: jnp.reshape(v, (1, -1))
    return pl.pallas_call(
        _body,
        out_shape=jax.ShapeDtypeStruct((_B, _N, _H), jnp.float32),
        compiler_params=pltpu.CompilerParams(
            vmem_limit_bytes=100 * 1024 * 1024),
    )(X, eps1, W1a, r(b1a), r(g1a), r(be1a), W1b, r(b1b), r(g1b), r(be1b),
      eps2, W2a, r(b2a), r(g2a), r(be2a), W2b, r(b2b), r(g2b), r(be2b))
